# baseline (speedup 1.0000x reference)
import functools

import jax
import jax.numpy as jnp
from jax import lax
from jax.experimental import pallas as pl
from jax.experimental.pallas import tpu as pltpu
from jax.experimental.pallas import tpu_sc as plsc

D = 64
SCALE = float(D) ** 0.5

NC = 2
NS = 16
NW = NC * NS
BT = 128


def _make_gather(S, NB):
    ntiles = S * (NB // BT)
    tpw = ntiles // NW
    nbt = NB // BT
    mesh = plsc.VectorSubcoreMesh(core_axis_name="c", subcore_axis_name="s")

    @functools.partial(
        pl.kernel,
        mesh=mesh,
        out_type=jax.ShapeDtypeStruct((S, 8, NB // BT, 8 * BT), jnp.float32),
        scratch_types=[
            pltpu.VMEM((BT,), jnp.int32),
            pltpu.VMEM((BT,), jnp.int32),
            pltpu.VMEM((BT, D), jnp.float32),
            pltpu.VMEM((BT, D), jnp.float32),
            pltpu.VMEM((8, 8 * BT), jnp.float32),
            pltpu.VMEM((8, 8 * BT), jnp.float32),
            pltpu.SemaphoreType.DMA,
            pltpu.SemaphoreType.DMA,
            pltpu.SemaphoreType.DMA,
            pltpu.SemaphoreType.DMA,
            pltpu.SemaphoreType.DMA,
            pltpu.SemaphoreType.DMA,
        ],
        compiler_params=pltpu.CompilerParams(
            use_tc_tiling_on_sc=False, needs_layout_passes=False),
    )
    def gather_k(table_hbm, xt_hbm, out_hbm,
                 idx0, idx1, rows0, rows1, t0, t1,
                 isem0, isem1, gsem0, gsem1, ssem0, ssem1):
        wid = lax.axis_index("s") * NC + lax.axis_index("c")
        tbase = wid * tpw
        last = tpw - 1

        idx_b = (idx0, idx1)
        rows_b = (rows0, rows1)
        t_b = (t0, t1)
        isem_b = (isem0, isem1)
        gsem_b = (gsem0, gsem1)
        ssem_b = (ssem0, ssem1)

        iot = lax.iota(jnp.int32, 16)
        chi = [(iot + 16 * k) // 8 for k in range(4)]
        clo = [((iot + 16 * k) % 8) * BT for k in range(4)]

        def coords(g):
            t = tbase + jnp.minimum(g, last)
            return t // nbt, t % nbt

        def istart(g, b):
            s, bi = coords(g)
            pltpu.make_async_copy(
                xt_hbm.at[s, pl.ds(bi * BT, BT)], idx_b[b], isem_b[b]).start()

        def iwait(b):
            pltpu.make_async_copy(
                xt_hbm.at[0, pl.ds(0, BT)], idx_b[b], isem_b[b]).wait()

        def gstart(b):
            pltpu.make_async_copy(
                table_hbm.at[idx_b[b]], rows_b[b], gsem_b[b]).start()

        def gwait(b):
            pltpu.make_async_copy(
                table_hbm.at[idx_b[b]], rows_b[b], gsem_b[b]).wait()

        def sstart(g, b):
            s, bi = coords(g)
            pltpu.make_async_copy(
                t_b[b], out_hbm.at[s, :, bi], ssem_b[b]).start()

        def swait(b):
            pltpu.make_async_copy(
                t_b[b], out_hbm.at[0, :, 0], ssem_b[b]).wait()

        def transpose_scale(rb, tb):
            rows, t = rows_b[rb], t_b[tb]

            def brow(r, c):
                vb = jnp.full((16,), r, jnp.int32)
                for k in range(4):
                    v = rows[r, pl.ds(16 * k, 16)] * SCALE
                    plsc.store_scatter(t, [chi[k], clo[k] + vb], v)
                return c

            lax.fori_loop(0, BT, brow, 0)

        istart(0, 0)
        iwait(0)
        gstart(0)
        istart(1, 1)

        def body(g, b0, b1):
            gwait(b0)
            iwait(b1)
            gstart(b1)

            @pl.when(g >= 2)
            def _():
                swait(b0)

            transpose_scale(b0, b0)
            sstart(g, b0)
            istart(g + 2, b0)

        def loop(j, carry):
            body(2 * j, 0, 1)
            body(2 * j + 1, 1, 0)
            return carry

        lax.fori_loop(0, tpw // 2, loop, 0)

        swait(0)
        swait(1)
        gwait(0)
        iwait(1)

    return gather_k


@jax.jit
def kernel(x, weight):
    b0, b1 = x.shape
    xt = x.T.astype(jnp.int32)
    out5 = _make_gather(b1, b0)(weight, xt)
    out5 = out5.reshape(b1, 8, b0 // BT, 8, BT)
    return out5.transpose(2, 4, 0, 1, 3).reshape(b0, b1, D)


# speedup vs baseline: 1.8083x; 1.8083x over previous
import functools

import jax
import jax.numpy as jnp
from jax import lax
from jax.experimental import pallas as pl
from jax.experimental.pallas import tpu as pltpu
from jax.experimental.pallas import tpu_sc as plsc

D = 64
SCALE = float(D) ** 0.5

NC = 2
NS = 16
NW = NC * NS
BT = 128


def _make_gather(S, NB):
    ntiles = S * (NB // BT)
    tpw = ntiles // NW
    nbt = NB // BT
    mesh = plsc.VectorSubcoreMesh(core_axis_name="c", subcore_axis_name="s")

    @functools.partial(
        pl.kernel,
        mesh=mesh,
        out_type=jax.ShapeDtypeStruct((S, 8, NB // BT, 8, BT), jnp.float32),
        scratch_types=[
            pltpu.VMEM((BT,), jnp.int32),
            pltpu.VMEM((BT,), jnp.int32),
            pltpu.VMEM((BT, D), jnp.float32),
            pltpu.VMEM((BT, D), jnp.float32),
            pltpu.VMEM((8, 8, BT + 1), jnp.float32),
            pltpu.VMEM((8, 8, BT + 1), jnp.float32),
            pltpu.SemaphoreType.DMA,
            pltpu.SemaphoreType.DMA,
            pltpu.SemaphoreType.DMA,
            pltpu.SemaphoreType.DMA,
            pltpu.SemaphoreType.DMA,
            pltpu.SemaphoreType.DMA,
        ],
        compiler_params=pltpu.CompilerParams(
            use_tc_tiling_on_sc=False, needs_layout_passes=False),
    )
    def gather_k(table_hbm, xt_hbm, out_hbm,
                 idx0, idx1, rows0, rows1, t0, t1,
                 isem0, isem1, gsem0, gsem1, ssem0, ssem1):
        wid = lax.axis_index("s") * NC + lax.axis_index("c")
        tbase = wid * tpw
        last = tpw - 1

        idx_b = (idx0, idx1)
        rows_b = (rows0, rows1)
        t_b = (t0, t1)
        isem_b = (isem0, isem1)
        gsem_b = (gsem0, gsem1)
        ssem_b = (ssem0, ssem1)

        iot = lax.iota(jnp.int32, 16)
        chi = [(iot + 16 * k) // 8 for k in range(4)]
        clo = [(iot + 16 * k) % 8 for k in range(4)]

        def coords(g):
            t = tbase + jnp.minimum(g, last)
            return t // nbt, t % nbt

        def istart(g, b):
            s, bi = coords(g)
            pltpu.make_async_copy(
                xt_hbm.at[s, pl.ds(bi * BT, BT)], idx_b[b], isem_b[b]).start()

        def iwait(b):
            pltpu.make_async_copy(
                xt_hbm.at[0, pl.ds(0, BT)], idx_b[b], isem_b[b]).wait()

        def gstart(b):
            pltpu.make_async_copy(
                table_hbm.at[idx_b[b]], rows_b[b], gsem_b[b]).start()

        def gwait(b):
            pltpu.make_async_copy(
                table_hbm.at[idx_b[b]], rows_b[b], gsem_b[b]).wait()

        def sstart(g, b):
            s, bi = coords(g)
            pltpu.make_async_copy(
                t_b[b].at[:, :, pl.ds(0, BT)], out_hbm.at[s, :, bi],
                ssem_b[b]).start()

        def swait(b):
            pltpu.make_async_copy(
                t_b[b].at[:, :, pl.ds(0, BT)], out_hbm.at[0, :, 0],
                ssem_b[b]).wait()

        def transpose_scale(rb, tb):
            rows, t = rows_b[rb], t_b[tb]

            def brow(r0, c):
                for u in range(4):
                    r = r0 * 4 + u
                    vb = jnp.full((16,), r, jnp.int32)
                    for k in range(4):
                        v = rows[r, pl.ds(16 * k, 16)] * SCALE
                        plsc.store_scatter(t, [chi[k], clo[k], vb], v)
                return c

            lax.fori_loop(0, BT // 4, brow, 0)

        istart(0, 0)
        iwait(0)
        gstart(0)
        istart(1, 1)

        def body(g, b0, b1):
            gwait(b0)
            iwait(b1)
            gstart(b1)

            @pl.when(g >= 2)
            def _():
                swait(b0)

            transpose_scale(b0, b0)
            sstart(g, b0)
            istart(g + 2, b0)

        def loop(j, carry):
            body(2 * j, 0, 1)
            body(2 * j + 1, 1, 0)
            return carry

        lax.fori_loop(0, tpw // 2, loop, 0)

        swait(0)
        swait(1)
        gwait(0)
        iwait(1)

    return gather_k


@jax.jit
def kernel(x, weight):
    b0, b1 = x.shape
    xt = x.T.astype(jnp.int32)
    out5 = _make_gather(b1, b0)(weight, xt)
    return out5.transpose(2, 4, 0, 1, 3).reshape(b0, b1, D)


# speedup vs baseline: 3.5276x; 1.9508x over previous
import functools

import jax
import jax.numpy as jnp
from jax import lax
from jax.experimental import pallas as pl
from jax.experimental.pallas import tpu as pltpu
from jax.experimental.pallas import tpu_sc as plsc

D = 64
SCALE = float(D) ** 0.5

NC = 2
NS = 16
NW = NC * NS
BT = 128


def _make_gather(S, NB):
    ntiles = S * (NB // BT)
    tpw = ntiles // NW
    nbt = NB // BT
    mesh = plsc.VectorSubcoreMesh(core_axis_name="c", subcore_axis_name="s")

    @functools.partial(
        pl.kernel,
        mesh=mesh,
        out_type=jax.ShapeDtypeStruct((S, 8, NB // BT, 8, BT), jnp.float32),
        scratch_types=[
            pltpu.VMEM((BT,), jnp.int32),
            pltpu.VMEM((BT,), jnp.int32),
            pltpu.VMEM((BT, D), jnp.float32),
            pltpu.VMEM((BT, D), jnp.float32),
            pltpu.VMEM((8, 8, BT + 1), jnp.float32),
            pltpu.VMEM((8, 8, BT + 1), jnp.float32),
            pltpu.SemaphoreType.DMA,
            pltpu.SemaphoreType.DMA,
            pltpu.SemaphoreType.DMA,
            pltpu.SemaphoreType.DMA,
            pltpu.SemaphoreType.DMA,
            pltpu.SemaphoreType.DMA,
        ],
        compiler_params=pltpu.CompilerParams(
            use_tc_tiling_on_sc=False, needs_layout_passes=False),
    )
    def gather_k(table_hbm, xt_hbm, out_hbm,
                 idx0, idx1, rows0, rows1, t0, t1,
                 isem0, isem1, gsem0, gsem1, ssem0, ssem1):
        wid = lax.axis_index("s") * NC + lax.axis_index("c")
        tbase = wid * tpw
        last = tpw - 1

        idx_b = (idx0, idx1)
        rows_b = (rows0, rows1)
        t_b = (t0, t1)
        isem_b = (isem0, isem1)
        gsem_b = (gsem0, gsem1)
        ssem_b = (ssem0, ssem1)

        iot = lax.iota(jnp.int32, 16)
        chi = [(iot + 16 * k) // 8 for k in range(4)]
        clo = [(iot + 16 * k) % 8 for k in range(4)]

        def coords(g):
            t = tbase + jnp.minimum(g, last)
            return t // nbt, t % nbt

        def istart(g, b):
            s, bi = coords(g)
            pltpu.make_async_copy(
                xt_hbm.at[s, pl.ds(bi * BT, BT)], idx_b[b], isem_b[b]).start()

        def iwait(b):
            pltpu.make_async_copy(
                xt_hbm.at[0, pl.ds(0, BT)], idx_b[b], isem_b[b]).wait()

        def gstart(b):
            pltpu.make_async_copy(
                table_hbm.at[idx_b[b]], rows_b[b], gsem_b[b]).start()

        def gwait(b):
            pltpu.make_async_copy(
                table_hbm.at[idx_b[b]], rows_b[b], gsem_b[b]).wait()

        def sstart(g, b):
            s, bi = coords(g)
            pltpu.make_async_copy(
                t_b[b].at[:, :, pl.ds(0, BT)], out_hbm.at[s, :, bi],
                ssem_b[b]).start()

        def swait(b):
            pltpu.make_async_copy(
                t_b[b].at[:, :, pl.ds(0, BT)], out_hbm.at[0, :, 0],
                ssem_b[b]).wait()

        def transpose_scale(rb, tb):
            rows, t = rows_b[rb], t_b[tb]

            @plsc.parallel_loop(0, BT, step=1, unroll=8)
            def brow(r):
                vb = jnp.full((16,), r, jnp.int32)
                for k in range(4):
                    v = rows[r, pl.ds(16 * k, 16)] * SCALE
                    plsc.store_scatter(t, [chi[k], clo[k], vb], v)

        istart(0, 0)
        iwait(0)
        gstart(0)
        istart(1, 1)

        def body(g, b0, b1):
            gwait(b0)
            iwait(b1)
            gstart(b1)

            @pl.when(g >= 2)
            def _():
                swait(b0)

            transpose_scale(b0, b0)
            sstart(g, b0)
            istart(g + 2, b0)

        def loop(j, carry):
            body(2 * j, 0, 1)
            body(2 * j + 1, 1, 0)
            return carry

        lax.fori_loop(0, tpw // 2, loop, 0)

        swait(0)
        swait(1)
        gwait(0)
        iwait(1)

    return gather_k


@jax.jit
def kernel(x, weight):
    b0, b1 = x.shape
    xt = x.T.astype(jnp.int32)
    out5 = _make_gather(b1, b0)(weight, xt)
    return out5.transpose(2, 4, 0, 1, 3).reshape(b0, b1, D)


# speedup vs baseline: 3.5432x; 1.0044x over previous
import functools

import jax
import jax.numpy as jnp
from jax import lax
from jax.experimental import pallas as pl
from jax.experimental.pallas import tpu as pltpu
from jax.experimental.pallas import tpu_sc as plsc

D = 64
SCALE = float(D) ** 0.5

NC = 2
NS = 16
NW = NC * NS
BT = 128


def _make_gather(S, NB):
    ntiles = S * (NB // BT)
    tpw = ntiles // NW
    nbt = NB // BT
    mesh = plsc.VectorSubcoreMesh(core_axis_name="c", subcore_axis_name="s")

    @functools.partial(
        pl.kernel,
        mesh=mesh,
        out_type=jax.ShapeDtypeStruct((S, 8, NB // BT, 8, BT), jnp.float32),
        scratch_types=[
            pltpu.VMEM((BT,), jnp.int32),
            pltpu.VMEM((BT,), jnp.int32),
            pltpu.VMEM((BT, D), jnp.float32),
            pltpu.VMEM((BT, D), jnp.float32),
            pltpu.VMEM((8, 8, BT + 1), jnp.float32),
            pltpu.VMEM((8, 8, BT + 1), jnp.float32),
            pltpu.SemaphoreType.DMA,
            pltpu.SemaphoreType.DMA,
            pltpu.SemaphoreType.DMA,
            pltpu.SemaphoreType.DMA,
            pltpu.SemaphoreType.DMA,
            pltpu.SemaphoreType.DMA,
        ],
        compiler_params=pltpu.CompilerParams(
            use_tc_tiling_on_sc=False, needs_layout_passes=False),
    )
    def gather_k(table_hbm, xt_hbm, out_hbm,
                 idx0, idx1, rows0, rows1, t0, t1,
                 isem0, isem1, gsem0, gsem1, ssem0, ssem1):
        wid = lax.axis_index("s") * NC + lax.axis_index("c")
        tbase = wid * tpw
        last = tpw - 1

        idx_b = (idx0, idx1)
        rows_b = (rows0, rows1)
        t_b = (t0, t1)
        isem_b = (isem0, isem1)
        gsem_b = (gsem0, gsem1)
        ssem_b = (ssem0, ssem1)

        iot = lax.iota(jnp.int32, 16)
        chi = [(iot + 16 * k) // 8 for k in range(4)]
        clo = [(iot + 16 * k) % 8 for k in range(4)]

        def coords(g):
            t = tbase + jnp.minimum(g, last)
            return t // nbt, t % nbt

        def istart(g, b):
            s, bi = coords(g)
            pltpu.make_async_copy(
                xt_hbm.at[s // 8, bi, s % 8], idx_b[b], isem_b[b]).start()

        def iwait(b):
            pltpu.make_async_copy(
                xt_hbm.at[0, 0, 0], idx_b[b], isem_b[b]).wait()

        def gstart(b):
            pltpu.make_async_copy(
                table_hbm.at[idx_b[b]], rows_b[b], gsem_b[b]).start()

        def gwait(b):
            pltpu.make_async_copy(
                table_hbm.at[idx_b[b]], rows_b[b], gsem_b[b]).wait()

        def sstart(g, b):
            s, bi = coords(g)
            pltpu.make_async_copy(
                t_b[b].at[:, :, pl.ds(0, BT)], out_hbm.at[s, :, bi],
                ssem_b[b]).start()

        def swait(b):
            pltpu.make_async_copy(
                t_b[b].at[:, :, pl.ds(0, BT)], out_hbm.at[0, :, 0],
                ssem_b[b]).wait()

        def transpose_scale(rb, tb):
            rows, t = rows_b[rb], t_b[tb]

            @plsc.parallel_loop(0, BT, step=1, unroll=8)
            def brow(r):
                vb = jnp.full((16,), r, jnp.int32)
                for k in range(4):
                    v = rows[r, pl.ds(16 * k, 16)] * SCALE
                    plsc.store_scatter(t, [chi[k], clo[k], vb], v)

        istart(0, 0)
        iwait(0)
        gstart(0)
        istart(1, 1)

        def body(g, b0, b1):
            gwait(b0)
            iwait(b1)
            gstart(b1)

            @pl.when(g >= 2)
            def _():
                swait(b0)

            transpose_scale(b0, b0)
            sstart(g, b0)
            istart(g + 2, b0)

        def loop(j, carry):
            body(2 * j, 0, 1)
            body(2 * j + 1, 1, 0)
            return carry

        lax.fori_loop(0, tpw // 2, loop, 0)

        swait(0)
        swait(1)
        gwait(0)
        iwait(1)

    return gather_k


@jax.jit
def kernel(x, weight):
    b0, b1 = x.shape
    x4 = x.T.reshape(b1 // 8, 8, b0 // BT, BT).transpose(0, 2, 1, 3)
    out5 = _make_gather(b1, b0)(weight, x4.astype(jnp.int32))
    return out5.transpose(2, 4, 0, 1, 3).reshape(b0, b1, D)


# speedup vs baseline: 3.6851x; 1.0401x over previous
import functools

import jax
import jax.numpy as jnp
from jax import lax
from jax.experimental import pallas as pl
from jax.experimental.pallas import tpu as pltpu
from jax.experimental.pallas import tpu_sc as plsc

D = 64
SCALE = float(D) ** 0.5

NC = 2
NS = 16
NW = NC * NS
BT = 128


def _fmt_body(w_ref, o_ref):
    xp = (w_ref[...].T * SCALE).reshape(o_ref.shape[0], 2, D)
    o_ref[:, 0:D] = xp[:, 0, :]
    o_ref[:, D:2 * D] = xp[:, 1, :]


def _format_table(weight):
    v = weight.shape[0]
    cols = 2048
    grid = pl.cdiv(v, cols)
    wt = weight.T
    return pl.pallas_call(
        _fmt_body,
        grid=(grid,),
        in_specs=[pl.BlockSpec((D, cols), lambda j: (0, j))],
        out_specs=pl.BlockSpec((cols // 2, 2 * D), lambda j: (j, 0)),
        out_shape=jax.ShapeDtypeStruct((v // 2, 2 * D), jnp.float32),
    )(wt)


def _make_gather(S, NB):
    ntiles = S * (NB // BT)
    tpw = ntiles // NW
    nbt = NB // BT
    mesh = plsc.VectorSubcoreMesh(core_axis_name="c", subcore_axis_name="s")

    @functools.partial(
        pl.kernel,
        mesh=mesh,
        out_type=jax.ShapeDtypeStruct((S, 8, NB // BT, 8, BT), jnp.float32),
        scratch_types=[
            pltpu.VMEM((BT,), jnp.int32),
            pltpu.VMEM((BT,), jnp.int32),
            pltpu.VMEM((BT, D), jnp.float32),
            pltpu.VMEM((BT, D), jnp.float32),
            pltpu.VMEM((8, 8, BT + 1), jnp.float32),
            pltpu.VMEM((8, 8, BT + 1), jnp.float32),
            pltpu.SemaphoreType.DMA,
            pltpu.SemaphoreType.DMA,
            pltpu.SemaphoreType.DMA,
            pltpu.SemaphoreType.DMA,
            pltpu.SemaphoreType.DMA,
            pltpu.SemaphoreType.DMA,
        ],
        compiler_params=pltpu.CompilerParams(
            use_tc_tiling_on_sc=False, needs_layout_passes=False),
    )
    def gather_k(table_hbm, xt_hbm, out_hbm,
                 idx0, idx1, rows0, rows1, t0, t1,
                 isem0, isem1, gsem0, gsem1, ssem0, ssem1):
        wid = lax.axis_index("s") * NC + lax.axis_index("c")
        tbase = wid * tpw
        last = tpw - 1

        idx_b = (idx0, idx1)
        rows_b = (rows0, rows1)
        t_b = (t0, t1)
        isem_b = (isem0, isem1)
        gsem_b = (gsem0, gsem1)
        ssem_b = (ssem0, ssem1)

        iot = lax.iota(jnp.int32, 16)
        chi = [(iot + 16 * k) // 8 for k in range(4)]
        clo = [(iot + 16 * k) % 8 for k in range(4)]

        def coords(g):
            t = tbase + jnp.minimum(g, last)
            return t // nbt, t % nbt

        def istart(g, b):
            s, bi = coords(g)
            pltpu.make_async_copy(
                xt_hbm.at[s // 8, bi, s % 8], idx_b[b], isem_b[b]).start()

        def iwait(b):
            pltpu.make_async_copy(
                xt_hbm.at[0, 0, 0], idx_b[b], isem_b[b]).wait()

        def gstart(b):
            pltpu.make_async_copy(
                table_hbm.at[idx_b[b]], rows_b[b], gsem_b[b]).start()

        def gwait(b):
            pltpu.make_async_copy(
                table_hbm.at[idx_b[b]], rows_b[b], gsem_b[b]).wait()

        def sstart(g, b):
            s, bi = coords(g)
            pltpu.make_async_copy(
                t_b[b].at[:, :, pl.ds(0, BT)], out_hbm.at[s, :, bi],
                ssem_b[b]).start()

        def swait(b):
            pltpu.make_async_copy(
                t_b[b].at[:, :, pl.ds(0, BT)], out_hbm.at[0, :, 0],
                ssem_b[b]).wait()

        def transpose_scale(rb, tb):
            rows, t = rows_b[rb], t_b[tb]

            @plsc.parallel_loop(0, BT, step=1, unroll=8)
            def brow(r):
                vb = jnp.full((16,), r, jnp.int32)
                for k in range(4):
                    v = rows[r, pl.ds(16 * k, 16)]
                    plsc.store_scatter(t, [chi[k], clo[k], vb], v)

        istart(0, 0)
        iwait(0)
        gstart(0)
        istart(1, 1)

        def body(g, b0, b1):
            gwait(b0)
            iwait(b1)
            gstart(b1)

            @pl.when(g >= 2)
            def _():
                swait(b0)

            transpose_scale(b0, b0)
            sstart(g, b0)
            istart(g + 2, b0)

        def loop(j, carry):
            body(2 * j, 0, 1)
            body(2 * j + 1, 1, 0)
            return carry

        lax.fori_loop(0, tpw // 2, loop, 0)

        swait(0)
        swait(1)
        gwait(0)
        iwait(1)

    return gather_k


@jax.jit
def kernel(x, weight):
    b0, b1 = x.shape
    x4 = x.T.reshape(b1 // 8, 8, b0 // BT, BT).transpose(0, 2, 1, 3)
    table = _format_table(weight).reshape(weight.shape)
    out5 = _make_gather(b1, b0)(table, x4.astype(jnp.int32))
    return out5.transpose(2, 4, 0, 1, 3).reshape(b0, b1, D)


# speedup vs baseline: 4.0138x; 1.0892x over previous
import functools

import jax
import jax.numpy as jnp
from jax import lax
from jax.experimental import pallas as pl
from jax.experimental.pallas import tpu as pltpu
from jax.experimental.pallas import tpu_sc as plsc

D = 64
SCALE = float(D) ** 0.5

NC = 2
NS = 16
NW = NC * NS
BT = 128


def _fmt_body(w_ref, o_ref):
    xp = (w_ref[...].T * SCALE).reshape(o_ref.shape[0], 2, D)
    o_ref[:, 0:D] = xp[:, 0, :]
    o_ref[:, D:2 * D] = xp[:, 1, :]


def _format_table(weight):
    v = weight.shape[0]
    cols = 8192
    grid = pl.cdiv(v, cols)
    wt = weight.T
    return pl.pallas_call(
        _fmt_body,
        grid=(grid,),
        in_specs=[pl.BlockSpec((D, cols), lambda j: (0, j))],
        out_specs=pl.BlockSpec((cols // 2, 2 * D), lambda j: (j, 0)),
        out_shape=jax.ShapeDtypeStruct((v // 2, 2 * D), jnp.float32),
    )(wt)


def _make_gather(S, NB):
    ntiles = S * (NB // BT)
    tpw = ntiles // NW
    nbt = NB // BT
    mesh = plsc.VectorSubcoreMesh(core_axis_name="c", subcore_axis_name="s")

    @functools.partial(
        pl.kernel,
        mesh=mesh,
        out_type=jax.ShapeDtypeStruct((S, 8, NB // BT, 8, BT), jnp.float32),
        scratch_types=[
            pltpu.VMEM((BT,), jnp.int32),
            pltpu.VMEM((BT,), jnp.int32),
            pltpu.VMEM((BT, D), jnp.float32),
            pltpu.VMEM((BT, D), jnp.float32),
            pltpu.VMEM((8, 8, BT + 1), jnp.float32),
            pltpu.VMEM((8, 8, BT + 1), jnp.float32),
            pltpu.SemaphoreType.DMA,
            pltpu.SemaphoreType.DMA,
            pltpu.SemaphoreType.DMA,
            pltpu.SemaphoreType.DMA,
            pltpu.SemaphoreType.DMA,
            pltpu.SemaphoreType.DMA,
        ],
        compiler_params=pltpu.CompilerParams(
            use_tc_tiling_on_sc=False, needs_layout_passes=False),
    )
    def gather_k(table_hbm, xt_hbm, out_hbm,
                 idx0, idx1, rows0, rows1, t0, t1,
                 isem0, isem1, gsem0, gsem1, ssem0, ssem1):
        wid = lax.axis_index("s") * NC + lax.axis_index("c")
        tbase = wid * tpw
        last = tpw - 1

        idx_b = (idx0, idx1)
        rows_b = (rows0, rows1)
        t_b = (t0, t1)
        isem_b = (isem0, isem1)
        gsem_b = (gsem0, gsem1)
        ssem_b = (ssem0, ssem1)

        iot = lax.iota(jnp.int32, 16)
        chi = [(iot + 16 * k) // 8 for k in range(4)]
        clo = [(iot + 16 * k) % 8 for k in range(4)]

        def coords(g):
            t = tbase + jnp.minimum(g, last)
            return t // nbt, t % nbt

        def istart(g, b):
            s, bi = coords(g)
            pltpu.make_async_copy(
                xt_hbm.at[s // 8, bi, s % 8], idx_b[b], isem_b[b]).start()

        def iwait(b):
            pltpu.make_async_copy(
                xt_hbm.at[0, 0, 0], idx_b[b], isem_b[b]).wait()

        def gstart(b):
            pltpu.make_async_copy(
                table_hbm.at[idx_b[b]], rows_b[b], gsem_b[b]).start()

        def gwait(b):
            pltpu.make_async_copy(
                table_hbm.at[idx_b[b]], rows_b[b], gsem_b[b]).wait()

        def sstart(g, b):
            s, bi = coords(g)
            pltpu.make_async_copy(
                t_b[b].at[:, :, pl.ds(0, BT)], out_hbm.at[s, :, bi],
                ssem_b[b]).start()

        def swait(b):
            pltpu.make_async_copy(
                t_b[b].at[:, :, pl.ds(0, BT)], out_hbm.at[0, :, 0],
                ssem_b[b]).wait()

        def transpose_scale(rb, tb):
            rows, t = rows_b[rb], t_b[tb]

            @plsc.parallel_loop(0, BT, step=1, unroll=16)
            def brow(r):
                vb = jnp.full((16,), r, jnp.int32)
                for k in range(4):
                    v = rows[r, pl.ds(16 * k, 16)]
                    plsc.store_scatter(t, [chi[k], clo[k], vb], v)

        istart(0, 0)
        iwait(0)
        gstart(0)
        istart(1, 1)

        def body(g, b0, b1):
            gwait(b0)
            iwait(b1)
            gstart(b1)

            @pl.when(g >= 2)
            def _():
                swait(b0)

            transpose_scale(b0, b0)
            sstart(g, b0)
            istart(g + 2, b0)

        def loop(j, carry):
            body(2 * j, 0, 1)
            body(2 * j + 1, 1, 0)
            return carry

        lax.fori_loop(0, tpw // 2, loop, 0)

        swait(0)
        swait(1)
        gwait(0)
        iwait(1)

    return gather_k


@jax.jit
def kernel(x, weight):
    b0, b1 = x.shape
    x4 = x.T.reshape(b1 // 8, 8, b0 // BT, BT).transpose(0, 2, 1, 3)
    table = _format_table(weight).reshape(weight.shape)
    out5 = _make_gather(b1, b0)(table, x4.astype(jnp.int32))
    return out5.transpose(2, 4, 0, 1, 3).reshape(b0, b1, D)
